# baseline (speedup 1.0000x reference)
import functools

import jax
import jax.numpy as jnp
from jax import lax
from jax.experimental import pallas as pl
from jax.experimental.pallas import tpu as pltpu
from jax.experimental.pallas import tpu_sc as plsc

N = 10000
E = 320000
H = 128

NC = 2
NS = 16
NW = NC * NS

N_PAD = 10240
CH = 128
EPT = 10112
NCHUNK = EPT // CH
E_PAD = NW * EPT
RPT = N_PAD // NS
K0 = 135
K1 = 2 * NCHUNK - K0
G0 = EPT
G1 = 2 * EPT - G0
GMAX = max(G0, G1)

_f32 = jnp.float32


def _mesh():
    return plsc.VectorSubcoreMesh(core_axis_name="c", subcore_axis_name="s",
                                  num_cores=NC, num_subcores=NS)


@functools.lru_cache(maxsize=None)
def _make_seg0():
    def body(u_hbm, v_hbm, src_hbm, dst_hbm, outd_hbm, outu_hbm, outv_hbm,
             u_v, v_v, src_v, dst_v, acc_d, acc_u, acc_v):
        c = lax.axis_index("c")
        s = lax.axis_index("s")
        wid = s * NC + c
        zvec = jnp.full((16,), 0.0, _f32)
        ones = jnp.full((16,), 1.0, _f32)

        pltpu.sync_copy(u_hbm, u_v)
        pltpu.sync_copy(v_hbm, v_v)

        @pl.when(c == 0)
        def _():
            pltpu.sync_copy(src_hbm.at[pl.ds(s * G0, G0)],
                            src_v.at[pl.ds(0, G0)])
            pltpu.sync_copy(dst_hbm.at[pl.ds(s * G0, G0)],
                            dst_v.at[pl.ds(0, G0)])

        @pl.when(c == 1)
        def _():
            pltpu.sync_copy(src_hbm.at[pl.ds(NS * G0 + s * G1, G1)],
                            src_v.at[pl.ds(0, G1)])
            pltpu.sync_copy(dst_hbm.at[pl.ds(NS * G0 + s * G1, G1)],
                            dst_v.at[pl.ds(0, G1)])

        def zero(i, carry):
            sl = pl.ds(i * 16, 16)
            acc_d[sl] = zvec
            acc_u[sl] = zvec
            acc_v[sl] = zvec
            return carry
        lax.fori_loop(0, N_PAD // 16, zero, 0)

        def step(i, carry):
            sl = pl.ds(i * 16, 16)
            sv = src_v[sl]
            dv = dst_v[sl]
            uu = plsc.load_gather(u_v, [sv])
            vv = plsc.load_gather(v_v, [sv])
            plsc.addupdate_scatter(acc_d, [dv], ones)
            plsc.addupdate_scatter(acc_u, [dv], uu)
            plsc.addupdate_scatter(acc_v, [dv], vv)
            return carry
        lax.fori_loop(0, jnp.where(c == 0, G0 // 16, G1 // 16), step, 0)

        base = wid * N_PAD
        pltpu.sync_copy(acc_d, outd_hbm.at[pl.ds(base, N_PAD)])
        pltpu.sync_copy(acc_u, outu_hbm.at[pl.ds(base, N_PAD)])
        pltpu.sync_copy(acc_v, outv_hbm.at[pl.ds(base, N_PAD)])

    return pl.kernel(
        body,
        out_type=(jax.ShapeDtypeStruct((NW * N_PAD,), _f32),
                  jax.ShapeDtypeStruct((NW * N_PAD,), _f32),
                  jax.ShapeDtypeStruct((NW * N_PAD,), _f32)),
        mesh=_mesh(),
        compiler_params=pltpu.CompilerParams(needs_layout_passes=False),
        scratch_types=[
            pltpu.VMEM((N_PAD,), _f32),
            pltpu.VMEM((N_PAD,), _f32),
            pltpu.VMEM((GMAX,), jnp.int32),
            pltpu.VMEM((GMAX,), jnp.int32),
            pltpu.VMEM((N_PAD,), _f32),
            pltpu.VMEM((N_PAD,), _f32),
            pltpu.VMEM((N_PAD,), _f32),
        ],
    )


@functools.lru_cache(maxsize=None)
def _make_seg1():
    def body(table_hbm, src_hbm, dst_hbm, out_hbm, src0_v, src1_v, dst0_v,
             dst1_v, rows0_v, rows1_v, acc_sh, sem0, sem1, ssem0, ssem1):
        c = lax.axis_index("c")
        s = lax.axis_index("s")
        zvec = jnp.full((16,), 0.0, _f32)
        srcs = (src0_v, src1_v)
        dsts = (dst0_v, dst1_v)
        rows = (rows0_v, rows1_v)
        sems = (sem0, sem1)
        ssems = (ssem0, ssem1)

        def zrow(i, carry):
            for j in range(H // 16):
                rows0_v[i, pl.ds(j * 16, 16)] = zvec
            return carry
        lax.fori_loop(0, CH, zrow, 0)
        r0 = s * RPT
        for b in range(RPT // CH):
            pltpu.sync_copy(rows0_v, acc_sh.at[pl.ds(r0 + b * CH, CH)])
        plsc.subcore_barrier()

        nck = jnp.where(c == 0, K0, K1)
        base = jnp.where(c == 0, s * (K0 * CH),
                         NS * K0 * CH + s * (K1 * CH))

        def fetch(j, b):
            off = base + j * CH
            pltpu.sync_copy(src_hbm.at[pl.ds(off, CH)], srcs[b])
            pltpu.sync_copy(dst_hbm.at[pl.ds(off, CH)], dsts[b])
            pltpu.async_copy(table_hbm.at[srcs[b]], rows[b], sems[b])

        fetch(0, 0)

        def step(i, carry):
            for b in range(2):
                j = i * 2 + b

                @pl.when(jnp.logical_and(j + 1 < nck, j >= 1))
                def _():
                    pltpu.make_async_copy(rows[1 - b],
                                          acc_sh.at[dsts[1 - b]],
                                          ssems[1 - b]).wait()

                @pl.when(j + 1 < nck)
                def _():
                    fetch(j + 1, 1 - b)

                @pl.when(j < nck)
                def _():
                    pltpu.make_async_copy(table_hbm.at[srcs[b]], rows[b],
                                          sems[b]).wait()
                    pltpu.async_copy(rows[b], acc_sh.at[dsts[b]], ssems[b],
                                     add=True)
            return carry
        lax.fori_loop(0, (nck + 1) // 2, step, 0)

        for b in range(2):
            pltpu.make_async_copy(rows[b], acc_sh.at[dsts[b]],
                                  ssems[b]).wait()
        plsc.subcore_barrier()

        for b in range(RPT // CH):
            sl = pl.ds(r0 + b * CH, CH)
            pltpu.sync_copy(acc_sh.at[sl], out_hbm.at[c, sl])

    return pl.kernel(
        body,
        out_type=jax.ShapeDtypeStruct((NC, N_PAD, H), _f32),
        mesh=_mesh(),
        scratch_types=[
            pltpu.VMEM((CH,), jnp.int32),
            pltpu.VMEM((CH,), jnp.int32),
            pltpu.VMEM((CH,), jnp.int32),
            pltpu.VMEM((CH,), jnp.int32),
            pltpu.VMEM((CH, H), _f32),
            pltpu.VMEM((CH, H), _f32),
            pltpu.VMEM_SHARED((N_PAD, H), _f32),
            pltpu.SemaphoreType.DMA,
            pltpu.SemaphoreType.DMA,
            pltpu.SemaphoreType.DMA,
            pltpu.SemaphoreType.DMA,
        ],
    )


BN1 = 1024


def _h1_body(pd_ref, pu_ref, pv_ref, uv_ref, ws_ref, wn_ref, b_ref, h1_ref):
    sd = jnp.sum(pd_ref[...], axis=0)[:, None]
    su = jnp.sum(pu_ref[...], axis=0)[:, None]
    sv = jnp.sum(pv_ref[...], axis=0)[:, None]
    inv = 1.0 / jnp.maximum(sd, 1.0)
    u = uv_ref[0][:, None]
    v = uv_ref[1][:, None]
    h = (u * ws_ref[0:1, :] + v * ws_ref[1:2, :]
         + su * inv * wn_ref[0:1, :] + sv * inv * wn_ref[1:2, :]
         + b_ref[0:1, :])
    h1_ref[...] = jnp.maximum(h, 0.0)


_h1_call = pl.pallas_call(
    _h1_body,
    grid=(N_PAD // BN1,),
    in_specs=[
        pl.BlockSpec((NW, BN1), lambda i: (0, i)),
        pl.BlockSpec((NW, BN1), lambda i: (0, i)),
        pl.BlockSpec((NW, BN1), lambda i: (0, i)),
        pl.BlockSpec((2, BN1), lambda i: (0, i)),
        pl.BlockSpec((2, H), lambda i: (0, 0)),
        pl.BlockSpec((2, H), lambda i: (0, 0)),
        pl.BlockSpec((1, H), lambda i: (0, 0)),
    ],
    out_specs=pl.BlockSpec((BN1, H), lambda i: (i, 0)),
    out_shape=jax.ShapeDtypeStruct((N_PAD, H), _f32),
)


BN2 = 1024
NB2 = N_PAD // BN2


def _h2_body(h1_ref, parts_ref, pd_ref, ws_ref, wn_ref, b_ref,
             wsc1_ref, bsc1_ref, wsc2t_ref, bsc2_ref, out_ref, acc):
    i = pl.program_id(0)
    sd = jnp.sum(pd_ref[...], axis=0)[:, None]
    inv = 1.0 / jnp.maximum(sd, 1.0)
    agg = (parts_ref[0] + parts_ref[1]) * inv
    h2 = jnp.dot(h1_ref[...], ws_ref[...], preferred_element_type=_f32)
    h2 = h2 + jnp.dot(agg, wn_ref[...], preferred_element_type=_f32)
    h2 = jnp.maximum(h2 + b_ref[0:1, :], 0.0)
    rows = i * BN2 + lax.broadcasted_iota(jnp.int32, (BN2, 1), 0)
    h2 = jnp.where(rows < N, h2, 0.0)
    part = jnp.sum(h2, axis=0, keepdims=True)

    @pl.when(i == 0)
    def _():
        acc[...] = part

    @pl.when(i > 0)
    def _():
        acc[...] = acc[...] + part

    @pl.when(i == NB2 - 1)
    def _():
        mean = acc[...] * (1.0 / N)
        hidden = jnp.maximum(
            jnp.dot(mean, wsc1_ref[...], preferred_element_type=_f32)
            + bsc1_ref[...], 0.0)
        out_ref[...] = (jnp.sum(hidden * wsc2t_ref[...], axis=1,
                                keepdims=True) + bsc2_ref[...])


_h2_call = pl.pallas_call(
    _h2_body,
    grid=(NB2,),
    in_specs=[
        pl.BlockSpec((BN2, H), lambda i: (i, 0)),
        pl.BlockSpec((2, BN2, H), lambda i: (0, i, 0)),
        pl.BlockSpec((NW, BN2), lambda i: (0, i)),
        pl.BlockSpec((H, H), lambda i: (0, 0)),
        pl.BlockSpec((H, H), lambda i: (0, 0)),
        pl.BlockSpec((1, H), lambda i: (0, 0)),
        pl.BlockSpec((H, H), lambda i: (0, 0)),
        pl.BlockSpec((1, H), lambda i: (0, 0)),
        pl.BlockSpec((1, H), lambda i: (0, 0)),
        pl.BlockSpec((1, 1), lambda i: (0, 0)),
    ],
    out_specs=pl.BlockSpec((1, 1), lambda i: (0, 0)),
    out_shape=jax.ShapeDtypeStruct((1, 1), _f32),
    scratch_shapes=[pltpu.VMEM((1, H), _f32)],
)


def kernel(edge_index, u_mask, v_mask, W_self0, W_neigh0, b0,
           W_self1, W_neigh1, b1, Wsc1, bsc1, Wsc2, bsc2):
    src = edge_index[0].astype(jnp.int32)
    dst = edge_index[1].astype(jnp.int32)
    pad = E_PAD - E
    src_p = jnp.concatenate([src, jnp.zeros((pad,), jnp.int32)])
    dst_p = jnp.concatenate([dst, jnp.full((pad,), N, jnp.int32)])
    uf = jnp.pad(u_mask.astype(_f32), (0, N_PAD - N))
    vf = jnp.pad(v_mask.astype(_f32), (0, N_PAD - N))

    pd, pu, pv = _make_seg0()(uf, vf, src_p, dst_p)
    pd = pd.reshape(NW, N_PAD)
    pu = pu.reshape(NW, N_PAD)
    pv = pv.reshape(NW, N_PAD)
    uv = jnp.stack([uf, vf])

    h1 = _h1_call(pd, pu, pv, uv, W_self0, W_neigh0, b0.reshape(1, H))
    parts = _make_seg1()(h1, src_p, dst_p)
    out = _h2_call(h1, parts, pd, W_self1, W_neigh1, b1.reshape(1, H),
                   Wsc1, bsc1.reshape(1, H), Wsc2.reshape(1, H),
                   bsc2.reshape(1, 1))
    return out.reshape(1)


# speedup vs baseline: 1.0450x; 1.0450x over previous
import functools

import jax
import jax.numpy as jnp
from jax import lax
from jax.experimental import pallas as pl
from jax.experimental.pallas import tpu as pltpu
from jax.experimental.pallas import tpu_sc as plsc

N = 10000
E = 320000
H = 128

NC = 2
NS = 16
NW = NC * NS

N_PAD = 10240
CH = 128
EPT = 10112
NCHUNK = EPT // CH
E_PAD = NW * EPT
RPT = N_PAD // NS
K0 = 125
K1 = 2 * NCHUNK - K0
G0 = EPT
G1 = 2 * EPT - G0
GMAX = max(G0, G1)

_f32 = jnp.float32


def _mesh():
    return plsc.VectorSubcoreMesh(core_axis_name="c", subcore_axis_name="s",
                                  num_cores=NC, num_subcores=NS)


@functools.lru_cache(maxsize=None)
def _make_seg0():
    def body(u_hbm, v_hbm, src_hbm, dst_hbm, outd_hbm, outu_hbm, outv_hbm,
             u_v, v_v, src_v, dst_v, acc_d, acc_u, acc_v):
        c = lax.axis_index("c")
        s = lax.axis_index("s")
        wid = s * NC + c
        zvec = jnp.full((16,), 0.0, _f32)
        ones = jnp.full((16,), 1.0, _f32)

        pltpu.sync_copy(u_hbm, u_v)
        pltpu.sync_copy(v_hbm, v_v)

        @pl.when(c == 0)
        def _():
            pltpu.sync_copy(src_hbm.at[pl.ds(s * G0, G0)],
                            src_v.at[pl.ds(0, G0)])
            pltpu.sync_copy(dst_hbm.at[pl.ds(s * G0, G0)],
                            dst_v.at[pl.ds(0, G0)])

        @pl.when(c == 1)
        def _():
            pltpu.sync_copy(src_hbm.at[pl.ds(NS * G0 + s * G1, G1)],
                            src_v.at[pl.ds(0, G1)])
            pltpu.sync_copy(dst_hbm.at[pl.ds(NS * G0 + s * G1, G1)],
                            dst_v.at[pl.ds(0, G1)])

        def zero(i, carry):
            sl = pl.ds(i * 16, 16)
            acc_d[sl] = zvec
            acc_u[sl] = zvec
            acc_v[sl] = zvec
            return carry
        lax.fori_loop(0, N_PAD // 16, zero, 0)

        def step(i, carry):
            sl = pl.ds(i * 16, 16)
            sv = src_v[sl]
            dv = dst_v[sl]
            uu = plsc.load_gather(u_v, [sv])
            vv = plsc.load_gather(v_v, [sv])
            plsc.addupdate_scatter(acc_d, [dv], ones)
            plsc.addupdate_scatter(acc_u, [dv], uu)
            plsc.addupdate_scatter(acc_v, [dv], vv)
            return carry
        lax.fori_loop(0, jnp.where(c == 0, G0 // 16, G1 // 16), step, 0)

        base = wid * N_PAD
        pltpu.sync_copy(acc_d, outd_hbm.at[pl.ds(base, N_PAD)])
        pltpu.sync_copy(acc_u, outu_hbm.at[pl.ds(base, N_PAD)])
        pltpu.sync_copy(acc_v, outv_hbm.at[pl.ds(base, N_PAD)])

    return pl.kernel(
        body,
        out_type=(jax.ShapeDtypeStruct((NW * N_PAD,), _f32),
                  jax.ShapeDtypeStruct((NW * N_PAD,), _f32),
                  jax.ShapeDtypeStruct((NW * N_PAD,), _f32)),
        mesh=_mesh(),
        compiler_params=pltpu.CompilerParams(needs_layout_passes=False),
        scratch_types=[
            pltpu.VMEM((N_PAD,), _f32),
            pltpu.VMEM((N_PAD,), _f32),
            pltpu.VMEM((GMAX,), jnp.int32),
            pltpu.VMEM((GMAX,), jnp.int32),
            pltpu.VMEM((N_PAD,), _f32),
            pltpu.VMEM((N_PAD,), _f32),
            pltpu.VMEM((N_PAD,), _f32),
        ],
    )


@functools.lru_cache(maxsize=None)
def _make_seg1():
    def body(table_hbm, src_hbm, dst_hbm, out_hbm, src0_v, src1_v, dst0_v,
             dst1_v, rows0_v, rows1_v, acc_sh, sem0, sem1, ssem0, ssem1):
        c = lax.axis_index("c")
        s = lax.axis_index("s")
        zvec = jnp.full((16,), 0.0, _f32)
        srcs = (src0_v, src1_v)
        dsts = (dst0_v, dst1_v)
        rows = (rows0_v, rows1_v)
        sems = (sem0, sem1)
        ssems = (ssem0, ssem1)

        def zrow(i, carry):
            for j in range(H // 16):
                rows0_v[i, pl.ds(j * 16, 16)] = zvec
            return carry
        lax.fori_loop(0, CH, zrow, 0)
        r0 = s * RPT
        for b in range(RPT // CH):
            pltpu.sync_copy(rows0_v, acc_sh.at[pl.ds(r0 + b * CH, CH)])
        plsc.subcore_barrier()

        nck = jnp.where(c == 0, K0, K1)
        base = jnp.where(c == 0, s * (K0 * CH),
                         NS * K0 * CH + s * (K1 * CH))

        def fetch(j, b):
            off = base + j * CH
            pltpu.sync_copy(src_hbm.at[pl.ds(off, CH)], srcs[b])
            pltpu.sync_copy(dst_hbm.at[pl.ds(off, CH)], dsts[b])
            pltpu.async_copy(table_hbm.at[srcs[b]], rows[b], sems[b])

        fetch(0, 0)

        def step(i, carry):
            for b in range(2):
                j = i * 2 + b

                @pl.when(jnp.logical_and(j + 1 < nck, j >= 1))
                def _():
                    pltpu.make_async_copy(rows[1 - b],
                                          acc_sh.at[dsts[1 - b]],
                                          ssems[1 - b]).wait()

                @pl.when(j + 1 < nck)
                def _():
                    fetch(j + 1, 1 - b)

                @pl.when(j < nck)
                def _():
                    pltpu.make_async_copy(table_hbm.at[srcs[b]], rows[b],
                                          sems[b]).wait()
                    pltpu.async_copy(rows[b], acc_sh.at[dsts[b]], ssems[b],
                                     add=True)
            return carry
        lax.fori_loop(0, (nck + 1) // 2, step, 0)

        for b in range(2):
            pltpu.make_async_copy(rows[b], acc_sh.at[dsts[b]],
                                  ssems[b]).wait()
        plsc.subcore_barrier()

        for b in range(RPT // CH):
            sl = pl.ds(r0 + b * CH, CH)
            pltpu.sync_copy(acc_sh.at[sl], out_hbm.at[c, sl])

    return pl.kernel(
        body,
        out_type=jax.ShapeDtypeStruct((NC, N_PAD, H), _f32),
        mesh=_mesh(),
        scratch_types=[
            pltpu.VMEM((CH,), jnp.int32),
            pltpu.VMEM((CH,), jnp.int32),
            pltpu.VMEM((CH,), jnp.int32),
            pltpu.VMEM((CH,), jnp.int32),
            pltpu.VMEM((CH, H), _f32),
            pltpu.VMEM((CH, H), _f32),
            pltpu.VMEM_SHARED((N_PAD, H), _f32),
            pltpu.SemaphoreType.DMA,
            pltpu.SemaphoreType.DMA,
            pltpu.SemaphoreType.DMA,
            pltpu.SemaphoreType.DMA,
        ],
    )


BN1 = 1024


def _h1_body(pd_ref, pu_ref, pv_ref, uv_ref, ws_ref, wn_ref, b_ref, h1_ref):
    sd = jnp.sum(pd_ref[...], axis=0)[:, None]
    su = jnp.sum(pu_ref[...], axis=0)[:, None]
    sv = jnp.sum(pv_ref[...], axis=0)[:, None]
    inv = 1.0 / jnp.maximum(sd, 1.0)
    u = uv_ref[0][:, None]
    v = uv_ref[1][:, None]
    h = (u * ws_ref[0:1, :] + v * ws_ref[1:2, :]
         + su * inv * wn_ref[0:1, :] + sv * inv * wn_ref[1:2, :]
         + b_ref[0:1, :])
    h1_ref[...] = jnp.maximum(h, 0.0)


_h1_call = pl.pallas_call(
    _h1_body,
    grid=(N_PAD // BN1,),
    in_specs=[
        pl.BlockSpec((NW, BN1), lambda i: (0, i)),
        pl.BlockSpec((NW, BN1), lambda i: (0, i)),
        pl.BlockSpec((NW, BN1), lambda i: (0, i)),
        pl.BlockSpec((2, BN1), lambda i: (0, i)),
        pl.BlockSpec((2, H), lambda i: (0, 0)),
        pl.BlockSpec((2, H), lambda i: (0, 0)),
        pl.BlockSpec((1, H), lambda i: (0, 0)),
    ],
    out_specs=pl.BlockSpec((BN1, H), lambda i: (i, 0)),
    out_shape=jax.ShapeDtypeStruct((N_PAD, H), _f32),
)


BN2 = 1024
NB2 = N_PAD // BN2


def _h2_body(h1_ref, parts_ref, pd_ref, ws_ref, wn_ref, b_ref,
             wsc1_ref, bsc1_ref, wsc2t_ref, bsc2_ref, out_ref, acc):
    i = pl.program_id(0)
    sd = jnp.sum(pd_ref[...], axis=0)[:, None]
    inv = 1.0 / jnp.maximum(sd, 1.0)
    agg = (parts_ref[0] + parts_ref[1]) * inv
    h2 = jnp.dot(h1_ref[...], ws_ref[...], preferred_element_type=_f32)
    h2 = h2 + jnp.dot(agg, wn_ref[...], preferred_element_type=_f32)
    h2 = jnp.maximum(h2 + b_ref[0:1, :], 0.0)
    rows = i * BN2 + lax.broadcasted_iota(jnp.int32, (BN2, 1), 0)
    h2 = jnp.where(rows < N, h2, 0.0)
    part = jnp.sum(h2, axis=0, keepdims=True)

    @pl.when(i == 0)
    def _():
        acc[...] = part

    @pl.when(i > 0)
    def _():
        acc[...] = acc[...] + part

    @pl.when(i == NB2 - 1)
    def _():
        mean = acc[...] * (1.0 / N)
        hidden = jnp.maximum(
            jnp.dot(mean, wsc1_ref[...], preferred_element_type=_f32)
            + bsc1_ref[...], 0.0)
        out_ref[...] = (jnp.sum(hidden * wsc2t_ref[...], axis=1,
                                keepdims=True) + bsc2_ref[...])


_h2_call = pl.pallas_call(
    _h2_body,
    grid=(NB2,),
    in_specs=[
        pl.BlockSpec((BN2, H), lambda i: (i, 0)),
        pl.BlockSpec((2, BN2, H), lambda i: (0, i, 0)),
        pl.BlockSpec((NW, BN2), lambda i: (0, i)),
        pl.BlockSpec((H, H), lambda i: (0, 0)),
        pl.BlockSpec((H, H), lambda i: (0, 0)),
        pl.BlockSpec((1, H), lambda i: (0, 0)),
        pl.BlockSpec((H, H), lambda i: (0, 0)),
        pl.BlockSpec((1, H), lambda i: (0, 0)),
        pl.BlockSpec((1, H), lambda i: (0, 0)),
        pl.BlockSpec((1, 1), lambda i: (0, 0)),
    ],
    out_specs=pl.BlockSpec((1, 1), lambda i: (0, 0)),
    out_shape=jax.ShapeDtypeStruct((1, 1), _f32),
    scratch_shapes=[pltpu.VMEM((1, H), _f32)],
)


def kernel(edge_index, u_mask, v_mask, W_self0, W_neigh0, b0,
           W_self1, W_neigh1, b1, Wsc1, bsc1, Wsc2, bsc2):
    src = edge_index[0].astype(jnp.int32)
    dst = edge_index[1].astype(jnp.int32)
    pad = E_PAD - E
    src_p = jnp.concatenate([src, jnp.zeros((pad,), jnp.int32)])
    dst_p = jnp.concatenate([dst, jnp.full((pad,), N, jnp.int32)])
    uf = jnp.pad(u_mask.astype(_f32), (0, N_PAD - N))
    vf = jnp.pad(v_mask.astype(_f32), (0, N_PAD - N))

    pd, pu, pv = _make_seg0()(uf, vf, src_p, dst_p)
    pd = pd.reshape(NW, N_PAD)
    pu = pu.reshape(NW, N_PAD)
    pv = pv.reshape(NW, N_PAD)
    uv = jnp.stack([uf, vf])

    h1 = _h1_call(pd, pu, pv, uv, W_self0, W_neigh0, b0.reshape(1, H))
    parts = _make_seg1()(h1, src_p, dst_p)
    out = _h2_call(h1, parts, pd, W_self1, W_neigh1, b1.reshape(1, H),
                   Wsc1, bsc1.reshape(1, H), Wsc2.reshape(1, H),
                   bsc2.reshape(1, 1))
    return out.reshape(1)
